# baseline (speedup 1.0000x reference)
import functools

import jax
import jax.numpy as jnp
from jax import lax
from jax.experimental import pallas as pl
from jax.experimental.pallas import tpu as pltpu
from jax.experimental.pallas import tpu_sc as plsc

NC = 2
NS = 16
NW = NC * NS
LANES = 16
C = 128


def _cdiv(a, b):
  return (a + b - 1) // b


def _make_sc_compact(N, NP, ET, ETP):
  mesh = plsc.VectorSubcoreMesh(
      core_axis_name="c", subcore_axis_name="s", num_cores=NC,
      num_subcores=NS)

  @functools.partial(
      pl.kernel,
      mesh=mesh,
      compiler_params=pltpu.CompilerParams(needs_layout_passes=False),
      out_type=[
          jax.ShapeDtypeStruct((2, NW, ETP), jnp.int32),
          jax.ShapeDtypeStruct((2, NW, ETP), jnp.int32),
          jax.ShapeDtypeStruct((2, NW, 16), jnp.int32),
      ],
      scratch_types=[
          pltpu.VMEM((ET,), jnp.int32),
          pltpu.VMEM((ET,), jnp.int32),
          pltpu.VMEM((ET,), jnp.int32),
          pltpu.VMEM((ETP + LANES,), jnp.int32),
          pltpu.VMEM((ETP + LANES,), jnp.int32),
          pltpu.VMEM((ETP + LANES,), jnp.int32),
          pltpu.VMEM((ETP + LANES,), jnp.int32),
          pltpu.VMEM((16,), jnp.int32),
      ],
  )
  def sc_compact(src_hbm, dst_hbm, attr_hbm, gl_hbm, sl_hbm, cnt_hbm,
                 edg_s, edg_d, edg_a, g1, s1, g2, s2, cbuf):
    cid = lax.axis_index("c")
    sid = lax.axis_index("s")
    wid = cid * NS + sid
    base = wid * ET

    pltpu.sync_copy(src_hbm.at[pl.ds(base, ET)], edg_s)
    pltpu.sync_copy(dst_hbm.at[pl.ds(base, ET)], edg_d)
    pltpu.sync_copy(attr_hbm.at[pl.ds(base, ET)], edg_a)

    lane = lax.iota(jnp.int32, LANES)
    trash = ETP

    def scan(i, cnts):
      c1, c2 = cnts
      s = edg_s[pl.ds(i * LANES, LANES)]
      d = edg_d[pl.ds(i * LANES, LANES)]
      a = edg_a[pl.ds(i * LANES, LANES)]
      m1 = a == 1
      m2 = a == 2
      i1 = m1.astype(jnp.int32)
      i2 = m2.astype(jnp.int32)
      pos1 = jnp.where(m1, c1 + plsc.cumsum(i1) - 1, trash + lane)
      pos2 = jnp.where(m2, c2 + plsc.cumsum(i2) - 1, trash + lane)
      plsc.store_scatter(g1, [pos1], s)
      plsc.store_scatter(s1, [pos1], d)
      plsc.store_scatter(g2, [pos2], s + NP)
      plsc.store_scatter(s2, [pos2], d)
      c1 = c1 + jnp.sum(i1)
      c2 = c2 + jnp.sum(i2)
      return (c1, c2)

    c1, c2 = lax.fori_loop(0, ET // LANES, scan, (jnp.int32(0), jnp.int32(0)))

    gdum = jnp.zeros((LANES,), jnp.int32)
    sdum = jnp.full((LANES,), N, jnp.int32)
    for t in range(2 * C // LANES):
      g1[pl.ds(c1 + t * LANES, LANES)] = gdum
      s1[pl.ds(c1 + t * LANES, LANES)] = sdum
      g2[pl.ds(c2 + t * LANES, LANES)] = gdum
      s2[pl.ds(c2 + t * LANES, LANES)] = sdum

    pltpu.sync_copy(g1.at[pl.ds(0, ETP)], gl_hbm.at[0, wid])
    pltpu.sync_copy(s1.at[pl.ds(0, ETP)], sl_hbm.at[0, wid])
    pltpu.sync_copy(g2.at[pl.ds(0, ETP)], gl_hbm.at[1, wid])
    pltpu.sync_copy(s2.at[pl.ds(0, ETP)], sl_hbm.at[1, wid])
    cbuf[...] = jnp.broadcast_to(c1, (16,)).astype(jnp.int32)
    pltpu.sync_copy(cbuf, cnt_hbm.at[0, wid])
    cbuf[...] = jnp.broadcast_to(c2, (16,)).astype(jnp.int32)
    pltpu.sync_copy(cbuf, cnt_hbm.at[1, wid])

  return sc_compact


def _make_sc_pass(nlists, N, NP, D, NCH, SEC):
  rows_per_tile = NP // NS
  NSEC = NCH // SEC

  mesh = plsc.VectorSubcoreMesh(
      core_axis_name="c", subcore_axis_name="s", num_cores=NC,
      num_subcores=NS)

  @functools.partial(
      pl.kernel,
      mesh=mesh,
      compiler_params=pltpu.CompilerParams(needs_layout_passes=False),
      out_type=jax.ShapeDtypeStruct((NC, NP, D), jnp.float32),
      scratch_types=[
          pltpu.VMEM((SEC, C), jnp.int32),
          pltpu.VMEM((SEC, C), jnp.int32),
          pltpu.VMEM((16,), jnp.int32),
          pltpu.VMEM((C, D), jnp.float32),
          pltpu.VMEM((C, D), jnp.float32),
          pltpu.VMEM_SHARED((NP, D), jnp.float32),
          pltpu.SemaphoreType.DMA,
          pltpu.SemaphoreType.DMA,
      ],
  )
  def sc_pass(gl_hbm, sl_hbm, cnt_hbm, table_hbm, zeros_hbm, out_hbm,
              gb, sb, cbuf, rows0, rows1, acc, sem0, sem1):
    cid = lax.axis_index("c")
    sid = lax.axis_index("s")
    wid = cid * NS + sid

    pltpu.sync_copy(zeros_hbm, acc.at[pl.ds(sid * rows_per_tile,
                                            rows_per_tile)])
    plsc.subcore_barrier()

    for l in range(nlists):
      pltpu.sync_copy(cnt_hbm.at[l, wid], cbuf)
      cnt = jnp.max(cbuf[...])
      nch = (cnt + C - 1) // C
      nchr = ((nch + 1) // 2) * 2

      def sec_step(s, carry):
        pltpu.sync_copy(gl_hbm.at[l, wid, pl.ds(s * SEC, SEC)], gb)
        pltpu.sync_copy(sl_hbm.at[l, wid, pl.ds(s * SEC, SEC)], sb)
        pairs = jnp.clip(nchr - s * SEC, 0, SEC) // 2

        def pair_step(p, carry2):
          j0 = 2 * p
          j1 = 2 * p + 1
          d0 = pltpu.async_copy(table_hbm.at[gb.at[j0]], rows0, sem0)
          d1 = pltpu.async_copy(table_hbm.at[gb.at[j1]], rows1, sem1)
          d0.wait()
          pltpu.sync_copy(rows0, acc.at[sb.at[j0]], add=True)
          d1.wait()
          pltpu.sync_copy(rows1, acc.at[sb.at[j1]], add=True)
          return carry2

        lax.fori_loop(0, pairs, pair_step, 0)
        return carry

      lax.fori_loop(0, NSEC, sec_step, 0)

    plsc.subcore_barrier()

    pltpu.sync_copy(acc.at[pl.ds(sid * rows_per_tile, rows_per_tile)],
                    out_hbm.at[cid, pl.ds(sid * rows_per_tile,
                                          rows_per_tile)])

  return sc_pass


def _mm_kernel(x, W, b, BM=1024):
  NP, D = x.shape

  def body(x_ref, w_ref, b_ref, o_ref):
    o_ref[...] = jnp.dot(x_ref[...], w_ref[...],
                         preferred_element_type=jnp.float32) + b_ref[...]

  return pl.pallas_call(
      body,
      grid=(NP // BM,),
      in_specs=[
          pl.BlockSpec((BM, D), lambda i: (i, 0)),
          pl.BlockSpec((D, D), lambda i: (0, 0)),
          pl.BlockSpec((1, D), lambda i: (0, 0)),
      ],
      out_specs=pl.BlockSpec((BM, D), lambda i: (i, 0)),
      out_shape=jax.ShapeDtypeStruct((NP, D), jnp.float32),
  )(x, W, b)


def _update_mm2_kernel(x, p, W1, b1, W2, b2, BM=1024):
  NP, D = x.shape

  def body(x_ref, p_ref, w1_ref, b1_ref, w2_ref, b2_ref,
           x1_ref, h1_ref, h2_ref):
    a = p_ref[0] + p_ref[1]
    x1 = x_ref[...] + jnp.maximum(a, 0.0)
    nrm = jnp.sqrt(jnp.sum(x1 * x1, axis=1, keepdims=True))
    x1 = x1 / jnp.maximum(nrm, 1e-12)
    x1_ref[...] = x1
    h1_ref[...] = jnp.dot(x1, w1_ref[...],
                          preferred_element_type=jnp.float32) + b1_ref[...]
    h2_ref[...] = jnp.dot(x1, w2_ref[...],
                          preferred_element_type=jnp.float32) + b2_ref[...]

  return pl.pallas_call(
      body,
      grid=(NP // BM,),
      in_specs=[
          pl.BlockSpec((BM, D), lambda i: (i, 0)),
          pl.BlockSpec((2, BM, D), lambda i: (0, i, 0)),
          pl.BlockSpec((D, D), lambda i: (0, 0)),
          pl.BlockSpec((1, D), lambda i: (0, 0)),
          pl.BlockSpec((D, D), lambda i: (0, 0)),
          pl.BlockSpec((1, D), lambda i: (0, 0)),
      ],
      out_specs=[
          pl.BlockSpec((BM, D), lambda i: (i, 0)),
          pl.BlockSpec((BM, D), lambda i: (i, 0)),
          pl.BlockSpec((BM, D), lambda i: (i, 0)),
      ],
      out_shape=[
          jax.ShapeDtypeStruct((NP, D), jnp.float32),
          jax.ShapeDtypeStruct((NP, D), jnp.float32),
          jax.ShapeDtypeStruct((NP, D), jnp.float32),
      ],
  )(x, p, W1, b1, W2, b2)


def _update_kernel(x, p, BM=1024):
  NP, D = x.shape

  def body(x_ref, p_ref, o_ref):
    a = p_ref[0] + p_ref[1]
    x1 = x_ref[...] + jnp.maximum(a, 0.0)
    nrm = jnp.sqrt(jnp.sum(x1 * x1, axis=1, keepdims=True))
    o_ref[...] = x1 / jnp.maximum(nrm, 1e-12)

  return pl.pallas_call(
      body,
      grid=(NP // BM,),
      in_specs=[
          pl.BlockSpec((BM, D), lambda i: (i, 0)),
          pl.BlockSpec((2, BM, D), lambda i: (0, i, 0)),
      ],
      out_specs=pl.BlockSpec((BM, D), lambda i: (i, 0)),
      out_shape=jax.ShapeDtypeStruct((NP, D), jnp.float32),
  )(x, p)


def kernel(x, edge_index, edge_attr, W0, b0, W1, b1, W2, b2):
  N, D = x.shape
  E = edge_index.shape[1]

  NP = _cdiv(N, 1024) * 1024
  EP = _cdiv(E, NW * C) * (NW * C)
  ET = EP // NW
  SEC = 32
  NCH = _cdiv(ET // C + 1, SEC) * SEC
  ETP = NCH * C

  src = jnp.pad(edge_index[0], (0, EP - E))
  dst = jnp.pad(edge_index[1], (0, EP - E))
  attr = jnp.pad(edge_attr, (0, EP - E))
  xp = jnp.pad(x, ((0, NP - N), (0, 0)))
  b0r = b0.reshape(1, D)
  b1r = b1.reshape(1, D)
  b2r = b2.reshape(1, D)

  zstripe = jnp.zeros((NP // NS, D), jnp.float32)

  compact = _make_sc_compact(N, NP, ET, ETP)
  sc0 = _make_sc_pass(1, N, NP, D, NCH, SEC)
  sc1 = _make_sc_pass(2, N, NP, D, NCH, SEC)

  gl, sl, cnts = compact(src, dst, attr)
  gl = gl.reshape(2, NW, NCH, C)
  sl = sl.reshape(2, NW, NCH, C)
  h0 = _mm_kernel(xp, W0, b0r)
  p0 = sc0(gl, sl, cnts, h0, zstripe)
  x1, h1, h2 = _update_mm2_kernel(xp, p0, W1, b1r, W2, b2r)
  h12 = jnp.concatenate([h1, h2], axis=0)
  p1 = sc1(gl, sl, cnts, h12, zstripe)
  out = _update_kernel(x1, p1)
  return out[:N]


# speedup vs baseline: 1.6461x; 1.6461x over previous
import functools

import jax
import jax.numpy as jnp
from jax import lax
from jax.experimental import pallas as pl
from jax.experimental.pallas import tpu as pltpu
from jax.experimental.pallas import tpu_sc as plsc

NC = 2
NS = 16
NW = NC * NS
LANES = 16
C = 128


def _cdiv(a, b):
  return (a + b - 1) // b


def _make_sc_compact(N, NP, ET, ETP):
  mesh = plsc.VectorSubcoreMesh(
      core_axis_name="c", subcore_axis_name="s", num_cores=NC,
      num_subcores=NS)

  @functools.partial(
      pl.kernel,
      mesh=mesh,
      compiler_params=pltpu.CompilerParams(needs_layout_passes=False),
      out_type=[
          jax.ShapeDtypeStruct((2, NW, ETP), jnp.int32),
          jax.ShapeDtypeStruct((2, NW, ETP), jnp.int32),
          jax.ShapeDtypeStruct((2, NW, 16), jnp.int32),
      ],
      scratch_types=[
          pltpu.VMEM((ET,), jnp.int32),
          pltpu.VMEM((ET,), jnp.int32),
          pltpu.VMEM((ET,), jnp.int32),
          pltpu.VMEM((ETP + LANES,), jnp.int32),
          pltpu.VMEM((ETP + LANES,), jnp.int32),
          pltpu.VMEM((ETP + LANES,), jnp.int32),
          pltpu.VMEM((ETP + LANES,), jnp.int32),
          pltpu.VMEM((16,), jnp.int32),
      ],
  )
  def sc_compact(src_hbm, dst_hbm, attr_hbm, gl_hbm, sl_hbm, cnt_hbm,
                 edg_s, edg_d, edg_a, g1, s1, g2, s2, cbuf):
    cid = lax.axis_index("c")
    sid = lax.axis_index("s")
    wid = cid * NS + sid
    base = wid * ET

    pltpu.sync_copy(src_hbm.at[pl.ds(base, ET)], edg_s)
    pltpu.sync_copy(dst_hbm.at[pl.ds(base, ET)], edg_d)
    pltpu.sync_copy(attr_hbm.at[pl.ds(base, ET)], edg_a)

    lane = lax.iota(jnp.int32, LANES)
    trash = ETP

    def scan(i, cnts):
      c1, c2 = cnts
      s = edg_s[pl.ds(i * LANES, LANES)]
      d = edg_d[pl.ds(i * LANES, LANES)]
      a = edg_a[pl.ds(i * LANES, LANES)]
      m1 = a == 1
      m2 = a == 2
      i1 = m1.astype(jnp.int32)
      i2 = m2.astype(jnp.int32)
      pos1 = jnp.where(m1, c1 + plsc.cumsum(i1) - 1, trash + lane)
      pos2 = jnp.where(m2, c2 + plsc.cumsum(i2) - 1, trash + lane)
      plsc.store_scatter(g1, [pos1], s)
      plsc.store_scatter(s1, [pos1], d)
      plsc.store_scatter(g2, [pos2], s + NP)
      plsc.store_scatter(s2, [pos2], d)
      c1 = c1 + jnp.sum(i1)
      c2 = c2 + jnp.sum(i2)
      return (c1, c2)

    c1, c2 = lax.fori_loop(0, ET // LANES, scan, (jnp.int32(0), jnp.int32(0)))

    gdum = jnp.zeros((LANES,), jnp.int32)
    sdum = jnp.full((LANES,), N, jnp.int32)
    for t in range(C // LANES):
      g1[pl.ds(c1 + t * LANES, LANES)] = gdum
      s1[pl.ds(c1 + t * LANES, LANES)] = sdum
      g2[pl.ds(c2 + t * LANES, LANES)] = gdum
      s2[pl.ds(c2 + t * LANES, LANES)] = sdum

    pltpu.sync_copy(g1.at[pl.ds(0, ETP)], gl_hbm.at[0, wid])
    pltpu.sync_copy(s1.at[pl.ds(0, ETP)], sl_hbm.at[0, wid])
    pltpu.sync_copy(g2.at[pl.ds(0, ETP)], gl_hbm.at[1, wid])
    pltpu.sync_copy(s2.at[pl.ds(0, ETP)], sl_hbm.at[1, wid])
    cbuf[...] = jnp.broadcast_to(c1, (16,)).astype(jnp.int32)
    pltpu.sync_copy(cbuf, cnt_hbm.at[0, wid])
    cbuf[...] = jnp.broadcast_to(c2, (16,)).astype(jnp.int32)
    pltpu.sync_copy(cbuf, cnt_hbm.at[1, wid])

  return sc_compact


def _make_sc_pass(nlists, N, NP, D, NCH):
  rows_per_tile = NP // NS

  mesh = plsc.VectorSubcoreMesh(
      core_axis_name="c", subcore_axis_name="s", num_cores=NC,
      num_subcores=NS)

  @functools.partial(
      pl.kernel,
      mesh=mesh,
      compiler_params=pltpu.CompilerParams(needs_layout_passes=False),
      out_type=jax.ShapeDtypeStruct((NC, NP, D), jnp.float32),
      scratch_types=[
          pltpu.VMEM((NCH, C), jnp.int32),
          pltpu.VMEM((NCH, C), jnp.int32),
          pltpu.VMEM((16,), jnp.int32),
          pltpu.VMEM((C, D), jnp.float32),
          pltpu.VMEM_SHARED((NP, D), jnp.float32),
          pltpu.SemaphoreType.DMA,
      ],
  )
  def sc_pass(gl_hbm, sl_hbm, cnt_hbm, table_hbm, zeros_hbm, out_hbm,
              gb, sb, cbuf, rows, acc, sem):
    cid = lax.axis_index("c")
    sid = lax.axis_index("s")
    wid = cid * NS + sid

    pltpu.sync_copy(zeros_hbm, acc.at[pl.ds(sid * rows_per_tile,
                                            rows_per_tile)])
    plsc.subcore_barrier()

    for l in range(nlists):
      pltpu.sync_copy(cnt_hbm.at[l, wid], cbuf)
      cnt = jnp.max(cbuf[...])
      nch = (cnt + C - 1) // C
      pltpu.sync_copy(gl_hbm.at[l, wid], gb)
      pltpu.sync_copy(sl_hbm.at[l, wid], sb)

      def step(j, carry):
        pltpu.async_copy(table_hbm.at[gb.at[j]], rows, sem).wait()
        pltpu.sync_copy(rows, acc.at[sb.at[j]], add=True)
        return carry

      lax.fori_loop(0, nch, step, 0)

    plsc.subcore_barrier()

    pltpu.sync_copy(acc.at[pl.ds(sid * rows_per_tile, rows_per_tile)],
                    out_hbm.at[cid, pl.ds(sid * rows_per_tile,
                                          rows_per_tile)])

  return sc_pass


def _mm_kernel(x, W, b, BM=1024):
  NP, D = x.shape

  def body(x_ref, w_ref, b_ref, o_ref):
    o_ref[...] = jnp.dot(x_ref[...], w_ref[...],
                         preferred_element_type=jnp.float32) + b_ref[...]

  return pl.pallas_call(
      body,
      grid=(NP // BM,),
      in_specs=[
          pl.BlockSpec((BM, D), lambda i: (i, 0)),
          pl.BlockSpec((D, D), lambda i: (0, 0)),
          pl.BlockSpec((1, D), lambda i: (0, 0)),
      ],
      out_specs=pl.BlockSpec((BM, D), lambda i: (i, 0)),
      out_shape=jax.ShapeDtypeStruct((NP, D), jnp.float32),
  )(x, W, b)


def _update_mm2_kernel(x, p, W1, b1, W2, b2, BM=1024):
  NP, D = x.shape

  def body(x_ref, p_ref, w1_ref, b1_ref, w2_ref, b2_ref,
           x1_ref, h1_ref, h2_ref):
    a = p_ref[0] + p_ref[1]
    x1 = x_ref[...] + jnp.maximum(a, 0.0)
    nrm = jnp.sqrt(jnp.sum(x1 * x1, axis=1, keepdims=True))
    x1 = x1 / jnp.maximum(nrm, 1e-12)
    x1_ref[...] = x1
    h1_ref[...] = jnp.dot(x1, w1_ref[...],
                          preferred_element_type=jnp.float32) + b1_ref[...]
    h2_ref[...] = jnp.dot(x1, w2_ref[...],
                          preferred_element_type=jnp.float32) + b2_ref[...]

  return pl.pallas_call(
      body,
      grid=(NP // BM,),
      in_specs=[
          pl.BlockSpec((BM, D), lambda i: (i, 0)),
          pl.BlockSpec((2, BM, D), lambda i: (0, i, 0)),
          pl.BlockSpec((D, D), lambda i: (0, 0)),
          pl.BlockSpec((1, D), lambda i: (0, 0)),
          pl.BlockSpec((D, D), lambda i: (0, 0)),
          pl.BlockSpec((1, D), lambda i: (0, 0)),
      ],
      out_specs=[
          pl.BlockSpec((BM, D), lambda i: (i, 0)),
          pl.BlockSpec((BM, D), lambda i: (i, 0)),
          pl.BlockSpec((BM, D), lambda i: (i, 0)),
      ],
      out_shape=[
          jax.ShapeDtypeStruct((NP, D), jnp.float32),
          jax.ShapeDtypeStruct((NP, D), jnp.float32),
          jax.ShapeDtypeStruct((NP, D), jnp.float32),
      ],
  )(x, p, W1, b1, W2, b2)


def _update_kernel(x, p, BM=1024):
  NP, D = x.shape

  def body(x_ref, p_ref, o_ref):
    a = p_ref[0] + p_ref[1]
    x1 = x_ref[...] + jnp.maximum(a, 0.0)
    nrm = jnp.sqrt(jnp.sum(x1 * x1, axis=1, keepdims=True))
    o_ref[...] = x1 / jnp.maximum(nrm, 1e-12)

  return pl.pallas_call(
      body,
      grid=(NP // BM,),
      in_specs=[
          pl.BlockSpec((BM, D), lambda i: (i, 0)),
          pl.BlockSpec((2, BM, D), lambda i: (0, i, 0)),
      ],
      out_specs=pl.BlockSpec((BM, D), lambda i: (i, 0)),
      out_shape=jax.ShapeDtypeStruct((NP, D), jnp.float32),
  )(x, p)


def kernel(x, edge_index, edge_attr, W0, b0, W1, b1, W2, b2):
  N, D = x.shape
  E = edge_index.shape[1]

  NP = _cdiv(N, 1024) * 1024
  EP = _cdiv(E, NW * C) * (NW * C)
  ET = EP // NW
  NCH = ET // C + 1
  ETP = NCH * C

  src = jnp.pad(edge_index[0], (0, EP - E))
  dst = jnp.pad(edge_index[1], (0, EP - E))
  attr = jnp.pad(edge_attr, (0, EP - E))
  xp = jnp.pad(x, ((0, NP - N), (0, 0)))
  b0r = b0.reshape(1, D)
  b1r = b1.reshape(1, D)
  b2r = b2.reshape(1, D)

  zstripe = jnp.zeros((NP // NS, D), jnp.float32)

  compact = _make_sc_compact(N, NP, ET, ETP)
  sc0 = _make_sc_pass(1, N, NP, D, NCH)
  sc1 = _make_sc_pass(2, N, NP, D, NCH)

  gl, sl, cnts = compact(src, dst, attr)
  gl = gl.reshape(2, NW, NCH, C)
  sl = sl.reshape(2, NW, NCH, C)
  h0 = _mm_kernel(xp, W0, b0r)
  p0 = sc0(gl, sl, cnts, h0, zstripe)
  x1, h1, h2 = _update_mm2_kernel(xp, p0, W1, b1r, W2, b2r)
  h12 = jnp.concatenate([h1, h2], axis=0)
  p1 = sc1(gl, sl, cnts, h12, zstripe)
  out = _update_kernel(x1, p1)
  return out[:N]


# speedup vs baseline: 1.6580x; 1.0072x over previous
import functools

import jax
import jax.numpy as jnp
from jax import lax
from jax.experimental import pallas as pl
from jax.experimental.pallas import tpu as pltpu
from jax.experimental.pallas import tpu_sc as plsc

NC = 2
NS = 16
NW = NC * NS
LANES = 16
C = 128


def _cdiv(a, b):
  return (a + b - 1) // b


def _make_sc_compact(N, NP, ET, ETP):
  mesh = plsc.VectorSubcoreMesh(
      core_axis_name="c", subcore_axis_name="s", num_cores=NC,
      num_subcores=NS)

  @functools.partial(
      pl.kernel,
      mesh=mesh,
      compiler_params=pltpu.CompilerParams(needs_layout_passes=False),
      out_type=[
          jax.ShapeDtypeStruct((2, NW, ETP), jnp.int32),
          jax.ShapeDtypeStruct((2, NW, ETP), jnp.int32),
          jax.ShapeDtypeStruct((2, NW, 16), jnp.int32),
      ],
      scratch_types=[
          pltpu.VMEM((ET,), jnp.int32),
          pltpu.VMEM((ET,), jnp.int32),
          pltpu.VMEM((ET,), jnp.int32),
          pltpu.VMEM((ETP + LANES,), jnp.int32),
          pltpu.VMEM((ETP + LANES,), jnp.int32),
          pltpu.VMEM((ETP + LANES,), jnp.int32),
          pltpu.VMEM((ETP + LANES,), jnp.int32),
          pltpu.VMEM((16,), jnp.int32),
      ],
  )
  def sc_compact(src_hbm, dst_hbm, attr_hbm, gl_hbm, sl_hbm, cnt_hbm,
                 edg_s, edg_d, edg_a, g1, s1, g2, s2, cbuf):
    cid = lax.axis_index("c")
    sid = lax.axis_index("s")
    wid = cid * NS + sid
    base = wid * ET

    pltpu.sync_copy(src_hbm.at[pl.ds(base, ET)], edg_s)
    pltpu.sync_copy(dst_hbm.at[pl.ds(base, ET)], edg_d)
    pltpu.sync_copy(attr_hbm.at[pl.ds(base, ET)], edg_a)

    lane = lax.iota(jnp.int32, LANES)
    trash = ETP

    def scan(i, cnts):
      c1, c2 = cnts
      s = edg_s[pl.ds(i * LANES, LANES)]
      d = edg_d[pl.ds(i * LANES, LANES)]
      a = edg_a[pl.ds(i * LANES, LANES)]
      m1 = a == 1
      m2 = a == 2
      i1 = m1.astype(jnp.int32)
      i2 = m2.astype(jnp.int32)
      pos1 = jnp.where(m1, c1 + plsc.cumsum(i1) - 1, trash + lane)
      pos2 = jnp.where(m2, c2 + plsc.cumsum(i2) - 1, trash + lane)
      plsc.store_scatter(g1, [pos1], s)
      plsc.store_scatter(s1, [pos1], d)
      plsc.store_scatter(g2, [pos2], s + NP)
      plsc.store_scatter(s2, [pos2], d)
      c1 = c1 + jnp.sum(i1)
      c2 = c2 + jnp.sum(i2)
      return (c1, c2)

    c1, c2 = lax.fori_loop(0, ET // LANES, scan, (jnp.int32(0), jnp.int32(0)))

    gdum = jnp.zeros((LANES,), jnp.int32)
    sdum = jnp.full((LANES,), N, jnp.int32)
    for t in range(C // LANES):
      g1[pl.ds(c1 + t * LANES, LANES)] = gdum
      s1[pl.ds(c1 + t * LANES, LANES)] = sdum
      g2[pl.ds(c2 + t * LANES, LANES)] = gdum
      s2[pl.ds(c2 + t * LANES, LANES)] = sdum

    pltpu.sync_copy(g1.at[pl.ds(0, ETP)], gl_hbm.at[0, wid])
    pltpu.sync_copy(s1.at[pl.ds(0, ETP)], sl_hbm.at[0, wid])
    pltpu.sync_copy(g2.at[pl.ds(0, ETP)], gl_hbm.at[1, wid])
    pltpu.sync_copy(s2.at[pl.ds(0, ETP)], sl_hbm.at[1, wid])
    cbuf[...] = jnp.broadcast_to(c1, (16,)).astype(jnp.int32)
    pltpu.sync_copy(cbuf, cnt_hbm.at[0, wid])
    cbuf[...] = jnp.broadcast_to(c2, (16,)).astype(jnp.int32)
    pltpu.sync_copy(cbuf, cnt_hbm.at[1, wid])

  return sc_compact


def _make_sc_pass(nlists, N, NP, D, NCH):
  rows_per_tile = NP // NS

  mesh = plsc.VectorSubcoreMesh(
      core_axis_name="c", subcore_axis_name="s", num_cores=NC,
      num_subcores=NS)

  @functools.partial(
      pl.kernel,
      mesh=mesh,
      compiler_params=pltpu.CompilerParams(needs_layout_passes=False),
      out_type=jax.ShapeDtypeStruct((NC, NP, D), jnp.float32),
      scratch_types=[
          pltpu.VMEM((NCH, C), jnp.int32),
          pltpu.VMEM((NCH, C), jnp.int32),
          pltpu.VMEM((16,), jnp.int32),
          pltpu.VMEM((C, D), jnp.float32),
          pltpu.VMEM_SHARED((NP, D), jnp.float32),
          pltpu.SemaphoreType.DMA,
          pltpu.SemaphoreType.DMA,
      ],
  )
  def sc_pass(gl_hbm, sl_hbm, cnt_hbm, table_hbm, zeros_hbm, out_hbm,
              gb, sb, cbuf, rows, acc, sem, zsem):
    cid = lax.axis_index("c")
    sid = lax.axis_index("s")
    wid = cid * NS + sid

    dz = pltpu.async_copy(zeros_hbm,
                          acc.at[pl.ds(sid * rows_per_tile, rows_per_tile)],
                          zsem)
    first = True
    for l in range(nlists):
      pltpu.sync_copy(cnt_hbm.at[l, wid], cbuf)
      cnt = jnp.max(cbuf[...])
      nch = (cnt + C - 1) // C
      pltpu.sync_copy(gl_hbm.at[l, wid], gb)
      pltpu.sync_copy(sl_hbm.at[l, wid], sb)
      if first:
        dz.wait()
        plsc.subcore_barrier()
        first = False

      def step(j, carry):
        pltpu.async_copy(table_hbm.at[gb.at[j]], rows, sem).wait()
        pltpu.sync_copy(rows, acc.at[sb.at[j]], add=True)
        return carry

      lax.fori_loop(0, nch, step, 0)

    plsc.subcore_barrier()

    pltpu.sync_copy(acc.at[pl.ds(sid * rows_per_tile, rows_per_tile)],
                    out_hbm.at[cid, pl.ds(sid * rows_per_tile,
                                          rows_per_tile)])

  return sc_pass


def _mm_kernel(x, W, b, BM=1024):
  NP, D = x.shape

  def body(x_ref, w_ref, b_ref, o_ref):
    o_ref[...] = jnp.dot(x_ref[...], w_ref[...],
                         preferred_element_type=jnp.float32) + b_ref[...]

  return pl.pallas_call(
      body,
      grid=(NP // BM,),
      in_specs=[
          pl.BlockSpec((BM, D), lambda i: (i, 0)),
          pl.BlockSpec((D, D), lambda i: (0, 0)),
          pl.BlockSpec((1, D), lambda i: (0, 0)),
      ],
      out_specs=pl.BlockSpec((BM, D), lambda i: (i, 0)),
      out_shape=jax.ShapeDtypeStruct((NP, D), jnp.float32),
  )(x, W, b)


def _update_mm2_kernel(x, p, W1, b1, W2, b2, BM=1024):
  NP, D = x.shape

  def body(x_ref, p_ref, w1_ref, b1_ref, w2_ref, b2_ref,
           x1_ref, h1_ref, h2_ref):
    a = p_ref[0] + p_ref[1]
    x1 = x_ref[...] + jnp.maximum(a, 0.0)
    nrm = jnp.sqrt(jnp.sum(x1 * x1, axis=1, keepdims=True))
    x1 = x1 / jnp.maximum(nrm, 1e-12)
    x1_ref[...] = x1
    h1_ref[...] = jnp.dot(x1, w1_ref[...],
                          preferred_element_type=jnp.float32) + b1_ref[...]
    h2_ref[...] = jnp.dot(x1, w2_ref[...],
                          preferred_element_type=jnp.float32) + b2_ref[...]

  return pl.pallas_call(
      body,
      grid=(NP // BM,),
      in_specs=[
          pl.BlockSpec((BM, D), lambda i: (i, 0)),
          pl.BlockSpec((2, BM, D), lambda i: (0, i, 0)),
          pl.BlockSpec((D, D), lambda i: (0, 0)),
          pl.BlockSpec((1, D), lambda i: (0, 0)),
          pl.BlockSpec((D, D), lambda i: (0, 0)),
          pl.BlockSpec((1, D), lambda i: (0, 0)),
      ],
      out_specs=[
          pl.BlockSpec((BM, D), lambda i: (i, 0)),
          pl.BlockSpec((BM, D), lambda i: (i, 0)),
          pl.BlockSpec((BM, D), lambda i: (i, 0)),
      ],
      out_shape=[
          jax.ShapeDtypeStruct((NP, D), jnp.float32),
          jax.ShapeDtypeStruct((NP, D), jnp.float32),
          jax.ShapeDtypeStruct((NP, D), jnp.float32),
      ],
  )(x, p, W1, b1, W2, b2)


def _update_kernel(x, p, BM=1024):
  NP, D = x.shape

  def body(x_ref, p_ref, o_ref):
    a = p_ref[0] + p_ref[1]
    x1 = x_ref[...] + jnp.maximum(a, 0.0)
    nrm = jnp.sqrt(jnp.sum(x1 * x1, axis=1, keepdims=True))
    o_ref[...] = x1 / jnp.maximum(nrm, 1e-12)

  return pl.pallas_call(
      body,
      grid=(NP // BM,),
      in_specs=[
          pl.BlockSpec((BM, D), lambda i: (i, 0)),
          pl.BlockSpec((2, BM, D), lambda i: (0, i, 0)),
      ],
      out_specs=pl.BlockSpec((BM, D), lambda i: (i, 0)),
      out_shape=jax.ShapeDtypeStruct((NP, D), jnp.float32),
  )(x, p)


def kernel(x, edge_index, edge_attr, W0, b0, W1, b1, W2, b2):
  N, D = x.shape
  E = edge_index.shape[1]

  NP = _cdiv(N, 1024) * 1024
  EP = _cdiv(E, NW * C) * (NW * C)
  ET = EP // NW
  NCH = ET // C + 1
  ETP = NCH * C

  src = jnp.pad(edge_index[0], (0, EP - E))
  dst = jnp.pad(edge_index[1], (0, EP - E))
  attr = jnp.pad(edge_attr, (0, EP - E))
  xp = jnp.pad(x, ((0, NP - N), (0, 0)))
  b0r = b0.reshape(1, D)
  b1r = b1.reshape(1, D)
  b2r = b2.reshape(1, D)

  zstripe = jnp.zeros((NP // NS, D), jnp.float32)

  compact = _make_sc_compact(N, NP, ET, ETP)
  sc0 = _make_sc_pass(1, N, NP, D, NCH)
  sc1 = _make_sc_pass(2, N, NP, D, NCH)

  gl, sl, cnts = compact(src, dst, attr)
  gl = gl.reshape(2, NW, NCH, C)
  sl = sl.reshape(2, NW, NCH, C)
  h0 = _mm_kernel(xp, W0, b0r)
  p0 = sc0(gl, sl, cnts, h0, zstripe)
  x1, h1, h2 = _update_mm2_kernel(xp, p0, W1, b1r, W2, b2r)
  h12 = jnp.concatenate([h1, h2], axis=0)
  p1 = sc1(gl, sl, cnts, h12, zstripe)
  out = _update_kernel(x1, p1)
  return out[:N]
